# baseline (speedup 1.0000x reference)
import functools

import jax
import jax.numpy as jnp
from jax import lax
from jax.experimental import pallas as pl
from jax.experimental.pallas import tpu as pltpu
from jax.experimental.pallas import tpu_sc as plsc

_VOCAB = 1000000
_D = 64
_B = 4096
_L = 200

_CHUNK = 100
_NC = 2
_NS = 16
_NW = _NC * _NS
_ITEMS = _B // _NW
_CHUNKS_PER_ITEM = _L // _CHUNK
_NCHUNK = _ITEMS * _CHUNKS_PER_ITEM


def _accum_chunk(buf, a0, a1, a2, a3):
    def body(k, carry):
        c0, c1, c2, c3 = carry
        c0 = c0 + buf[k, pl.ds(0, 16)]
        c1 = c1 + buf[k, pl.ds(16, 16)]
        c2 = c2 + buf[k, pl.ds(32, 16)]
        c3 = c3 + buf[k, pl.ds(48, 16)]
        return c0, c1, c2, c3

    return lax.fori_loop(0, _CHUNK, body, (a0, a1, a2, a3), unroll=20)


def _pooled_embedding(x2, weights):
    mesh = plsc.VectorSubcoreMesh(core_axis_name="c", subcore_axis_name="s")

    @functools.partial(
        pl.kernel,
        mesh=mesh,
        out_type=jax.ShapeDtypeStruct((_B, _D), jnp.float32),
        compiler_params=pltpu.CompilerParams(use_tc_tiling_on_sc=False),
        scratch_types=[
            pltpu.VMEM((_NCHUNK, _CHUNK), jnp.int32),
            pltpu.VMEM((_CHUNK, _D), jnp.float32),
            pltpu.VMEM((_CHUNK, _D), jnp.float32),
            pltpu.VMEM((_ITEMS, _D), jnp.float32),
            pltpu.SemaphoreType.DMA,
            pltpu.SemaphoreType.DMA,
        ],
    )
    def k(x_hbm, w_hbm, out_hbm, idx_v, buf_a, buf_b, out_v, sem_a, sem_b):
        wid = lax.axis_index("s") * _NC + lax.axis_index("c")
        pltpu.sync_copy(x_hbm.at[pl.ds(wid * _NCHUNK, _NCHUNK)], idx_v)

        pltpu.async_copy(w_hbm.at[idx_v.at[0]], buf_a, sem_a)
        pltpu.async_copy(w_hbm.at[idx_v.at[1]], buf_b, sem_b)

        inv_l = jnp.float32(1.0 / _L)
        zero = jnp.zeros((16,), jnp.float32)

        def item(i, _):
            pltpu.make_async_copy(w_hbm.at[idx_v.at[0]], buf_a, sem_a).wait()
            a0, a1, a2, a3 = _accum_chunk(buf_a, zero, zero, zero, zero)

            @pl.when(i < _ITEMS - 1)
            def _():
                pltpu.async_copy(w_hbm.at[idx_v.at[2 * i + 2]], buf_a, sem_a)

            pltpu.make_async_copy(w_hbm.at[idx_v.at[1]], buf_b, sem_b).wait()
            a0, a1, a2, a3 = _accum_chunk(buf_b, a0, a1, a2, a3)

            @pl.when(i < _ITEMS - 1)
            def _():
                pltpu.async_copy(w_hbm.at[idx_v.at[2 * i + 3]], buf_b, sem_b)

            out_v[i, pl.ds(0, 16)] = a0 * inv_l
            out_v[i, pl.ds(16, 16)] = a1 * inv_l
            out_v[i, pl.ds(32, 16)] = a2 * inv_l
            out_v[i, pl.ds(48, 16)] = a3 * inv_l
            return 0

        lax.fori_loop(0, _ITEMS, item, 0)

        pltpu.sync_copy(out_v, out_hbm.at[pl.ds(wid * _ITEMS, _ITEMS)])

    return k(x2, weights)


def kernel(x, weights):
    x2 = x.astype(jnp.int32).reshape(-1, _CHUNK)
    return _pooled_embedding(x2, weights)


# speedup vs baseline: 1.0588x; 1.0588x over previous
import functools

import jax
import jax.numpy as jnp
from jax import lax
from jax.experimental import pallas as pl
from jax.experimental.pallas import tpu as pltpu
from jax.experimental.pallas import tpu_sc as plsc

_VOCAB = 1000000
_D = 64
_B = 4096
_L = 200

_NC = 2
_NS = 16
_NW = _NC * _NS
_ITEMS = _B // _NW
_PAIRS = _ITEMS // 2


def _accum_item(buf, a0, a1, a2, a3):
    def body(k, carry):
        c0, c1, c2, c3 = carry
        c0 = c0 + buf[k, pl.ds(0, 16)]
        c1 = c1 + buf[k, pl.ds(16, 16)]
        c2 = c2 + buf[k, pl.ds(32, 16)]
        c3 = c3 + buf[k, pl.ds(48, 16)]
        return c0, c1, c2, c3

    return lax.fori_loop(0, _L, body, (a0, a1, a2, a3), unroll=8)


def _pooled_embedding(x, weights):
    mesh = plsc.VectorSubcoreMesh(core_axis_name="c", subcore_axis_name="s")

    @functools.partial(
        pl.kernel,
        mesh=mesh,
        out_type=jax.ShapeDtypeStruct((_B, _D), jnp.float32),
        compiler_params=pltpu.CompilerParams(use_tc_tiling_on_sc=False),
        scratch_types=[
            pltpu.VMEM((_ITEMS, _L), jnp.int32),
            pltpu.VMEM((_L, _D), jnp.float32),
            pltpu.VMEM((_L, _D), jnp.float32),
            pltpu.VMEM((_ITEMS, _D), jnp.float32),
            pltpu.SemaphoreType.DMA,
            pltpu.SemaphoreType.DMA,
        ],
    )
    def k(x_hbm, w_hbm, out_hbm, idx_v, buf_a, buf_b, out_v, sem_a, sem_b):
        wid = lax.axis_index("s") * _NC + lax.axis_index("c")
        pltpu.sync_copy(x_hbm.at[pl.ds(wid * _ITEMS, _ITEMS)], idx_v)

        pltpu.async_copy(w_hbm.at[idx_v.at[0]], buf_a, sem_a)
        pltpu.async_copy(w_hbm.at[idx_v.at[1]], buf_b, sem_b)

        inv_l = jnp.float32(1.0 / _L)
        zero = jnp.zeros((16,), jnp.float32)

        def pair(p, _):
            i = 2 * p
            pltpu.make_async_copy(w_hbm.at[idx_v.at[0]], buf_a, sem_a).wait()
            a0, a1, a2, a3 = _accum_item(buf_a, zero, zero, zero, zero)

            @pl.when(p < _PAIRS - 1)
            def _():
                pltpu.async_copy(w_hbm.at[idx_v.at[i + 2]], buf_a, sem_a)

            out_v[i, pl.ds(0, 16)] = a0 * inv_l
            out_v[i, pl.ds(16, 16)] = a1 * inv_l
            out_v[i, pl.ds(32, 16)] = a2 * inv_l
            out_v[i, pl.ds(48, 16)] = a3 * inv_l

            pltpu.make_async_copy(w_hbm.at[idx_v.at[1]], buf_b, sem_b).wait()
            b0, b1, b2, b3 = _accum_item(buf_b, zero, zero, zero, zero)

            @pl.when(p < _PAIRS - 1)
            def _():
                pltpu.async_copy(w_hbm.at[idx_v.at[i + 3]], buf_b, sem_b)

            out_v[i + 1, pl.ds(0, 16)] = b0 * inv_l
            out_v[i + 1, pl.ds(16, 16)] = b1 * inv_l
            out_v[i + 1, pl.ds(32, 16)] = b2 * inv_l
            out_v[i + 1, pl.ds(48, 16)] = b3 * inv_l
            return 0

        lax.fori_loop(0, _PAIRS, pair, 0)

        pltpu.sync_copy(out_v, out_hbm.at[pl.ds(wid * _ITEMS, _ITEMS)])

    return k(x, weights)


def kernel(x, weights):
    return _pooled_embedding(x.astype(jnp.int32), weights)
